# baseline (speedup 1.0000x reference)
import functools

import jax
import jax.numpy as jnp
import numpy as np
from jax import lax
from jax.experimental import pallas as pl
from jax.experimental.pallas import tpu as pltpu
from jax.experimental.pallas import tpu_sc as plsc

_ATOM_DIMS = [119, 5, 12, 12, 10, 6, 6, 2, 2]
_BOND_DIMS = [5, 6, 2]
_AOFF = np.concatenate([[0], np.cumsum(_ATOM_DIMS)[:-1]]).astype(np.int32)
_BOFF = np.concatenate([[0], np.cumsum(_BOND_DIMS)[:-1]]).astype(np.int32)

_N = 10000
_E = 320000
_H = 128
_L = 3
_G = 128

_NC, _NS, _NW = 2, 16, 32
_CE = 128
_NP = 10240
_CH = 80
_KE = 126
_EP = _NW * _KE * _CH
_NA = 10112
_RT = _NA // _NS

_mesh = plsc.VectorSubcoreMesh(
    core_axis_name="c", subcore_axis_name="s", num_cores=_NC, num_subcores=_NS
)


def _add_rows(dst, src, rows):
    @pl.loop(0, rows)
    def _(r):
        for j in range(_H // 16):
            sl = pl.ds(j * 16, 16)
            dst[r, sl] = dst[r, sl] + src[r, sl]


_REP = 16

@functools.partial(
    pl.kernel,
    out_type=jax.ShapeDtypeStruct((_NP, _H), jnp.float32),
    mesh=_mesh,
    scratch_types=[
        pltpu.VMEM((_CE,), jnp.int32),
        pltpu.VMEM((_CE, _H), jnp.float32),
        pltpu.VMEM((_CE, _H), jnp.float32),
    ],
)
def _encode(aidx, atab, z0, idx_v, acc_v, fld_v):
    c = lax.axis_index("c")
    s = lax.axis_index("s")
    w = c * _NS + s

    @pl.loop(0, 3)
    def _(k):
        cid = w + k * _NW

        @pl.when(cid < _NP // _CE)
        def _():
            base = cid * _CE
            pltpu.sync_copy(aidx.at[pl.ds(base, _CE)], idx_v)
            pltpu.sync_copy(atab.at[idx_v], acc_v)
            for f in range(1, 9):
                pltpu.sync_copy(aidx.at[pl.ds(f * _NP + base, _CE)], idx_v)
                pltpu.sync_copy(atab.at[idx_v], fld_v)
                _add_rows(acc_v, fld_v, _CE)
            pltpu.sync_copy(acc_v, z0.at[pl.ds(base, _CE)])


@functools.partial(
    pl.kernel,
    out_type=jax.ShapeDtypeStruct((_NC, _NA, _H), jnp.float32),
    mesh=_mesh,
    scratch_types=[
        pltpu.VMEM_SHARED((_NA, _H), jnp.float32),
        pltpu.VMEM((2, _CH), jnp.int32),
        pltpu.VMEM((2, _CH), jnp.int32),
        pltpu.VMEM((2, _CH), jnp.int32),
        pltpu.VMEM((2, _CH), jnp.int32),
        pltpu.VMEM((_CH, _H), jnp.float32),
        pltpu.VMEM((_CH, _H), jnp.float32),
        pltpu.VMEM((_CH, _H), jnp.float32),
        pltpu.VMEM((_CH, _H), jnp.float32),
        pltpu.SemaphoreType.DMA,
        pltpu.SemaphoreType.DMA,
        pltpu.SemaphoreType.DMA,
        pltpu.SemaphoreType.DMA,
        pltpu.SemaphoreType.DMA,
        pltpu.SemaphoreType.DMA,
    ],
)
def _message(z, cidx, src, dst, ctab, zero, part, acc,
             src_v, cidx_v, dst_v, dst_sc, zb0, zb1, eb0, eb1,
             si0, si1, sg0, sg1, ss0, ss1):
    c = lax.axis_index("c")
    s = lax.axis_index("s")
    w = c * _NS + s
    zb = (zb0, zb1)
    eb = (eb0, eb1)
    si = (si0, si1)
    sg = (sg0, sg1)
    ss = (ss0, ss1)

    def issue_idx(k, b):
        base = (w * _KE + k) * _CH
        pltpu.async_copy(src.at[pl.ds(base, _CH)], src_v.at[b], si[b])
        pltpu.async_copy(cidx.at[pl.ds(base, _CH)], cidx_v.at[b], si[b])
        pltpu.async_copy(dst.at[pl.ds(base, _CH)], dst_v.at[b], si[b])

    def wait_idx(b):
        for ref in (src_v, cidx_v, dst_v):
            pltpu.make_async_copy(src.at[pl.ds(0, _CH)], ref.at[b], si[b]).wait()

    def issue_gathers(b):
        pltpu.async_copy(z.at[src_v.at[b]], zb[b], sg[b])
        pltpu.async_copy(ctab.at[cidx_v.at[b]], eb[b], sg[b])

    def wait_gathers(b):
        pltpu.make_async_copy(z.at[pl.ds(0, _CH)], zb[b], sg[b]).wait()
        pltpu.make_async_copy(z.at[pl.ds(0, _CH)], eb[b], sg[b]).wait()

    def issue_scatter(b):
        for j in range(_CH // 16):
            sl = pl.ds(j * 16, 16)
            dst_sc[b, sl] = dst_v[b, sl]
        pltpu.async_copy(zb[b], acc.at[dst_sc.at[b]], ss[b], add=True)

    def wait_scatter(b):
        pltpu.make_async_copy(zb[b], acc.at[pl.ds(0, _CH)], ss[b]).wait()

    def relu_add(b):
        @pl.loop(0, _CH, unroll=8)
        def _(r):
            for j in range(_H // 16):
                sl = pl.ds(j * 16, 16)
                zb[b][r, sl] = jnp.maximum(zb[b][r, sl] + eb[b][r, sl], 0.0)

    pltpu.sync_copy(zero, acc.at[pl.ds(s * _RT, _RT)])
    plsc.subcore_barrier()

    issue_idx(0, 0)
    wait_idx(0)
    issue_gathers(0)
    issue_idx(1, 1)

    @pl.loop(0, _KE // 2)
    def _(t):
        k0 = t * 2
        wait_gathers(0)
        wait_idx(1)

        @pl.when(t > 0)
        def _():
            wait_scatter(1)

        issue_gathers(1)
        relu_add(0)
        issue_scatter(0)

        @pl.when(k0 + 2 < _KE)
        def _():
            issue_idx(k0 + 2, 0)

        wait_gathers(1)
        relu_add(1)

        @pl.when(k0 + 2 < _KE)
        def _():
            wait_idx(0)
            wait_scatter(0)
            issue_gathers(0)

        issue_scatter(1)

        @pl.when(k0 + 3 < _KE)
        def _():
            issue_idx(k0 + 3, 1)

    wait_scatter(0)
    wait_scatter(1)
    plsc.subcore_barrier()
    pltpu.sync_copy(acc.at[pl.ds(s * _RT, _RT)], part.at[c, pl.ds(s * _RT, _RT)])


@functools.partial(
    pl.kernel,
    out_type=jax.ShapeDtypeStruct((_L * _G, _H), jnp.float32),
    mesh=_mesh,
    scratch_types=[
        pltpu.VMEM_SHARED((_L * _G, _H), jnp.float32),
        pltpu.VMEM((80,), jnp.int32),
        pltpu.VMEM((80, _H), jnp.float32),
    ],
)
def _pool(z0, z1, z2, boff, zero, g, acc, idx_v, buf):
    c = lax.axis_index("c")
    s = lax.axis_index("s")
    nrows = _L * _G // _NS

    @pl.when(c == 0)
    def _():
        pltpu.sync_copy(zero.at[pl.ds(0, nrows)], acc.at[pl.ds(s * nrows, nrows)])

    plsc.subcore_barrier()

    @pl.when(c == 0)
    def _():
        for l, zref in enumerate((z0, z1, z2)):
            @pl.loop(0, 8)
            def _(k, l=l, zref=zref):
                cid = s + k * _NS

                @pl.when(cid < _N // 80)
                def _():
                    row = cid * 80
                    pltpu.sync_copy(boff.at[pl.ds(l * _N + row, 80)], idx_v)
                    pltpu.sync_copy(zref.at[pl.ds(row, 80)], buf)
                    pltpu.sync_copy(buf, acc.at[idx_v], add=True)

    plsc.subcore_barrier()

    @pl.when(c == 0)
    def _():
        pltpu.sync_copy(acc.at[pl.ds(s * nrows, nrows)], g.at[pl.ds(s * nrows, nrows)])


def _mlp_body(final, z_ref, p_ref, w1, b1, g1, be1, w2, b2, gbn, bbn, out):
    h = z_ref[...] + p_ref[0, 0:_N, :] + p_ref[1, 0:_N, :]
    h1 = jnp.dot(h, w1[...], preferred_element_type=jnp.float32) + b1[...]
    m = jnp.mean(h1, axis=0, keepdims=True)
    d = h1 - m
    v = jnp.mean(d * d, axis=0, keepdims=True)
    h1 = d * (g1[...] * lax.rsqrt(v + 1e-5)) + be1[...]
    h1 = jnp.maximum(h1, 0.0)
    h2 = jnp.dot(h1, w2[...], preferred_element_type=jnp.float32) + b2[...]
    m2 = jnp.mean(h2, axis=0, keepdims=True)
    d2 = h2 - m2
    v2 = jnp.mean(d2 * d2, axis=0, keepdims=True)
    zn = d2 * (gbn[...] * lax.rsqrt(v2 + 1e-5)) + bbn[...]
    if not final:
        zn = jnp.maximum(zn, 0.0)
    out[...] = zn


def _mlp(final):
    return pl.pallas_call(
        functools.partial(_mlp_body, final),
        out_shape=jax.ShapeDtypeStruct((_N, _H), jnp.float32),
    )


@jax.jit
def kernel(x, edge_index, edge_attr, batch, atom_table, bond_table,
           W1, b1, g1, be1, W2, b2, gbn, bbn):
    nrep = jnp.arange(_N, dtype=jnp.int32) % _REP
    aidx = (x.astype(jnp.int32) + jnp.asarray(_AOFF)[None, :]).T
    aidx = aidx + (nrep * np.int32(sum(_ATOM_DIMS)))[None, :]
    aidx = jnp.pad(aidx, ((0, 0), (0, _NP - _N))).reshape(-1)
    atab_rep = jnp.tile(atom_table, (_REP, 1))

    bt0 = bond_table[_BOFF[0]:_BOFF[0] + _BOND_DIMS[0]]
    bt1 = bond_table[_BOFF[1]:_BOFF[1] + _BOND_DIMS[1]]
    bt2 = bond_table[_BOFF[2]:_BOFF[2] + _BOND_DIMS[2]]
    ctab = (bt0[:, None, None, :] + bt1[None, :, None, :]
            + bt2[None, None, :, :]).reshape(-1, _H)
    ctab_rep = jnp.tile(ctab, (_REP, 1))
    ea = edge_attr.astype(jnp.int32)
    ncomb = np.int32(_BOND_DIMS[1] * _BOND_DIMS[2])
    bidx = (ea[:, 0] * ncomb + ea[:, 1] * np.int32(_BOND_DIMS[2]) + ea[:, 2]
            + (jnp.arange(_E, dtype=jnp.int32) % _REP) * np.int32(60))
    bidx = jnp.pad(bidx, (0, _EP - _E))
    src = jnp.pad(edge_index[0].astype(jnp.int32), (0, _EP - _E))
    dst = jnp.pad(edge_index[1].astype(jnp.int32), (0, _EP - _E),
                  constant_values=_N)
    boff = (batch.astype(jnp.int32)[None, :]
            + (jnp.arange(_L, dtype=jnp.int32) * _G)[:, None]).reshape(-1)
    zero = jnp.zeros((_RT, _H), jnp.float32)

    z0f = _encode(aidx, atab_rep)
    z = z0f[:_N]

    zs = []
    for l in range(_L):
        part = _message(z, bidx, src, dst, ctab_rep, zero)
        z = _mlp(l == _L - 1)(
            z, part, W1[l], b1[l][None, :], g1[l][None, :], be1[l][None, :],
            W2[l], b2[l][None, :], gbn[l][None, :], bbn[l][None, :])
        zs.append(z)

    g = _pool(zs[0], zs[1], zs[2], boff, zero)
    z_cat = jnp.concatenate(zs, axis=1)
    g_cat = jnp.concatenate([g[l * _G:(l + 1) * _G] for l in range(_L)], axis=1)
    return (z_cat, g_cat)


# speedup vs baseline: 1.2877x; 1.2877x over previous
import functools

import jax
import jax.numpy as jnp
import numpy as np
from jax import lax
from jax.experimental import pallas as pl
from jax.experimental.pallas import tpu as pltpu
from jax.experimental.pallas import tpu_sc as plsc

_ATOM_DIMS = [119, 5, 12, 12, 10, 6, 6, 2, 2]
_BOND_DIMS = [5, 6, 2]
_AOFF = np.concatenate([[0], np.cumsum(_ATOM_DIMS)[:-1]]).astype(np.int32)
_BOFF = np.concatenate([[0], np.cumsum(_BOND_DIMS)[:-1]]).astype(np.int32)

_N = 10000
_E = 320000
_H = 128
_L = 3
_G = 128

_NC, _NS, _NW = 2, 16, 32
_CE = 128
_NP = 10240
_CH = 80
_KE = 126
_EP = _NW * _KE * _CH
_NA = 10112
_RT = _NA // _NS

_mesh = plsc.VectorSubcoreMesh(
    core_axis_name="c", subcore_axis_name="s", num_cores=_NC, num_subcores=_NS
)


def _add_rows(dst, src, rows):
    @pl.loop(0, rows)
    def _(r):
        for j in range(_H // 16):
            sl = pl.ds(j * 16, 16)
            dst[r, sl] = dst[r, sl] + src[r, sl]


_REP = 16

@functools.partial(
    pl.kernel,
    out_type=jax.ShapeDtypeStruct((_NP, _H), jnp.float32),
    mesh=_mesh,
    scratch_types=[
        pltpu.VMEM((_CE,), jnp.int32),
        pltpu.VMEM((_CE, _H), jnp.float32),
        pltpu.VMEM((_CE, _H), jnp.float32),
    ],
)
def _encode(aidx, atab, z0, idx_v, acc_v, fld_v):
    c = lax.axis_index("c")
    s = lax.axis_index("s")
    w = c * _NS + s

    @pl.loop(0, 3)
    def _(k):
        cid = w + k * _NW

        @pl.when(cid < _NP // _CE)
        def _():
            base = cid * _CE
            pltpu.sync_copy(aidx.at[pl.ds(base, _CE)], idx_v)
            pltpu.sync_copy(atab.at[idx_v], acc_v)
            for f in range(1, 9):
                pltpu.sync_copy(aidx.at[pl.ds(f * _NP + base, _CE)], idx_v)
                pltpu.sync_copy(atab.at[idx_v], fld_v)
                _add_rows(acc_v, fld_v, _CE)
            pltpu.sync_copy(acc_v, z0.at[pl.ds(base, _CE)])


@functools.partial(
    pl.kernel,
    out_type=jax.ShapeDtypeStruct((_NC, _NA, _H), jnp.float32),
    mesh=_mesh,
    scratch_types=[
        pltpu.VMEM_SHARED((_NA, _H), jnp.float32),
        pltpu.VMEM((2, _CH), jnp.int32),
        pltpu.VMEM((2, _CH), jnp.int32),
        pltpu.VMEM((2, _CH), jnp.int32),
        pltpu.VMEM((2, _CH), jnp.int32),
        pltpu.VMEM((_CH, _H), jnp.float32),
        pltpu.VMEM((_CH, _H), jnp.float32),
        pltpu.VMEM((_CH, _H), jnp.float32),
        pltpu.VMEM((_CH, _H), jnp.float32),
        pltpu.SemaphoreType.DMA,
        pltpu.SemaphoreType.DMA,
        pltpu.SemaphoreType.DMA,
        pltpu.SemaphoreType.DMA,
        pltpu.SemaphoreType.DMA,
        pltpu.SemaphoreType.DMA,
    ],
)
def _message(z, cidx, src, dst, ctab, zero, part, acc,
             src_v, cidx_v, dst_v, dst_sc, zb0, zb1, eb0, eb1,
             si0, si1, sg0, sg1, ss0, ss1):
    c = lax.axis_index("c")
    s = lax.axis_index("s")
    w = c * _NS + s
    zb = (zb0, zb1)
    eb = (eb0, eb1)
    si = (si0, si1)
    sg = (sg0, sg1)
    ss = (ss0, ss1)

    def issue_idx(k, b):
        base = (w * _KE + k) * _CH
        pltpu.async_copy(src.at[pl.ds(base, _CH)], src_v.at[b], si[b])
        pltpu.async_copy(cidx.at[pl.ds(base, _CH)], cidx_v.at[b], si[b])
        pltpu.async_copy(dst.at[pl.ds(base, _CH)], dst_v.at[b], si[b])

    def wait_idx(b):
        for ref in (src_v, cidx_v, dst_v):
            pltpu.make_async_copy(src.at[pl.ds(0, _CH)], ref.at[b], si[b]).wait()

    def issue_gathers(b):
        pltpu.async_copy(z.at[src_v.at[b]], zb[b], sg[b])
        pltpu.async_copy(ctab.at[cidx_v.at[b]], eb[b], sg[b])

    def wait_gathers(b):
        pltpu.make_async_copy(z.at[pl.ds(0, _CH)], zb[b], sg[b]).wait()
        pltpu.make_async_copy(z.at[pl.ds(0, _CH)], eb[b], sg[b]).wait()

    def issue_scatter(b):
        for j in range(_CH // 16):
            sl = pl.ds(j * 16, 16)
            dst_sc[b, sl] = dst_v[b, sl]
        pltpu.async_copy(zb[b], acc.at[dst_sc.at[b]], ss[b], add=True)

    def wait_scatter(b):
        pltpu.make_async_copy(zb[b], acc.at[pl.ds(0, _CH)], ss[b]).wait()

    def relu_add(b):
        @functools.partial(plsc.parallel_loop, 0, _CH, unroll=2)
        def _(r):
            for j in range(_H // 16):
                sl = pl.ds(j * 16, 16)
                zb[b][r, sl] = jnp.maximum(zb[b][r, sl] + eb[b][r, sl], 0.0)

    pltpu.sync_copy(zero, acc.at[pl.ds(s * _RT, _RT)])
    plsc.subcore_barrier()

    issue_idx(0, 0)
    wait_idx(0)
    issue_gathers(0)
    issue_idx(1, 1)

    @pl.loop(0, _KE // 2)
    def _(t):
        k0 = t * 2
        wait_gathers(0)
        wait_idx(1)

        @pl.when(t > 0)
        def _():
            wait_scatter(1)

        issue_gathers(1)
        relu_add(0)
        issue_scatter(0)

        @pl.when(k0 + 2 < _KE)
        def _():
            issue_idx(k0 + 2, 0)

        wait_gathers(1)
        relu_add(1)

        @pl.when(k0 + 2 < _KE)
        def _():
            wait_idx(0)
            wait_scatter(0)
            issue_gathers(0)

        issue_scatter(1)

        @pl.when(k0 + 3 < _KE)
        def _():
            issue_idx(k0 + 3, 1)

    wait_scatter(0)
    wait_scatter(1)
    plsc.subcore_barrier()
    pltpu.sync_copy(acc.at[pl.ds(s * _RT, _RT)], part.at[c, pl.ds(s * _RT, _RT)])


@functools.partial(
    pl.kernel,
    out_type=jax.ShapeDtypeStruct((_L * _G, _H), jnp.float32),
    mesh=_mesh,
    scratch_types=[
        pltpu.VMEM_SHARED((_L * _G, _H), jnp.float32),
        pltpu.VMEM((80,), jnp.int32),
        pltpu.VMEM((80, _H), jnp.float32),
    ],
)
def _pool(z0, z1, z2, boff, zero, g, acc, idx_v, buf):
    c = lax.axis_index("c")
    s = lax.axis_index("s")
    nrows = _L * _G // _NS

    @pl.when(c == 0)
    def _():
        pltpu.sync_copy(zero.at[pl.ds(0, nrows)], acc.at[pl.ds(s * nrows, nrows)])

    plsc.subcore_barrier()

    @pl.when(c == 0)
    def _():
        for l, zref in enumerate((z0, z1, z2)):
            @pl.loop(0, 8)
            def _(k, l=l, zref=zref):
                cid = s + k * _NS

                @pl.when(cid < _N // 80)
                def _():
                    row = cid * 80
                    pltpu.sync_copy(boff.at[pl.ds(l * _N + row, 80)], idx_v)
                    pltpu.sync_copy(zref.at[pl.ds(row, 80)], buf)
                    pltpu.sync_copy(buf, acc.at[idx_v], add=True)

    plsc.subcore_barrier()

    @pl.when(c == 0)
    def _():
        pltpu.sync_copy(acc.at[pl.ds(s * nrows, nrows)], g.at[pl.ds(s * nrows, nrows)])


def _mlp_body(final, z_ref, p_ref, w1, b1, g1, be1, w2, b2, gbn, bbn, out):
    h = z_ref[...] + p_ref[0, 0:_N, :] + p_ref[1, 0:_N, :]
    h1 = jnp.dot(h, w1[...], preferred_element_type=jnp.float32) + b1[...]
    m = jnp.mean(h1, axis=0, keepdims=True)
    d = h1 - m
    v = jnp.mean(d * d, axis=0, keepdims=True)
    h1 = d * (g1[...] * lax.rsqrt(v + 1e-5)) + be1[...]
    h1 = jnp.maximum(h1, 0.0)
    h2 = jnp.dot(h1, w2[...], preferred_element_type=jnp.float32) + b2[...]
    m2 = jnp.mean(h2, axis=0, keepdims=True)
    d2 = h2 - m2
    v2 = jnp.mean(d2 * d2, axis=0, keepdims=True)
    zn = d2 * (gbn[...] * lax.rsqrt(v2 + 1e-5)) + bbn[...]
    if not final:
        zn = jnp.maximum(zn, 0.0)
    out[...] = zn


def _mlp(final):
    return pl.pallas_call(
        functools.partial(_mlp_body, final),
        out_shape=jax.ShapeDtypeStruct((_N, _H), jnp.float32),
    )


@jax.jit
def kernel(x, edge_index, edge_attr, batch, atom_table, bond_table,
           W1, b1, g1, be1, W2, b2, gbn, bbn):
    nrep = jnp.arange(_N, dtype=jnp.int32) % _REP
    aidx = (x.astype(jnp.int32) + jnp.asarray(_AOFF)[None, :]).T
    aidx = aidx + (nrep * np.int32(sum(_ATOM_DIMS)))[None, :]
    aidx = jnp.pad(aidx, ((0, 0), (0, _NP - _N))).reshape(-1)
    atab_rep = jnp.tile(atom_table, (_REP, 1))

    bt0 = bond_table[_BOFF[0]:_BOFF[0] + _BOND_DIMS[0]]
    bt1 = bond_table[_BOFF[1]:_BOFF[1] + _BOND_DIMS[1]]
    bt2 = bond_table[_BOFF[2]:_BOFF[2] + _BOND_DIMS[2]]
    ctab = (bt0[:, None, None, :] + bt1[None, :, None, :]
            + bt2[None, None, :, :]).reshape(-1, _H)
    ctab_rep = jnp.tile(ctab, (_REP, 1))
    ea = edge_attr.astype(jnp.int32)
    ncomb = np.int32(_BOND_DIMS[1] * _BOND_DIMS[2])
    bidx = (ea[:, 0] * ncomb + ea[:, 1] * np.int32(_BOND_DIMS[2]) + ea[:, 2]
            + (jnp.arange(_E, dtype=jnp.int32) % _REP) * np.int32(60))
    bidx = jnp.pad(bidx, (0, _EP - _E))
    src = jnp.pad(edge_index[0].astype(jnp.int32), (0, _EP - _E))
    dst = jnp.pad(edge_index[1].astype(jnp.int32), (0, _EP - _E),
                  constant_values=_N)
    boff = (batch.astype(jnp.int32)[None, :]
            + (jnp.arange(_L, dtype=jnp.int32) * _G)[:, None]).reshape(-1)
    zero = jnp.zeros((_RT, _H), jnp.float32)

    z0f = _encode(aidx, atab_rep)
    z = z0f[:_N]

    zs = []
    for l in range(_L):
        part = _message(z, bidx, src, dst, ctab_rep, zero)
        z = _mlp(l == _L - 1)(
            z, part, W1[l], b1[l][None, :], g1[l][None, :], be1[l][None, :],
            W2[l], b2[l][None, :], gbn[l][None, :], bbn[l][None, :])
        zs.append(z)

    g = _pool(zs[0], zs[1], zs[2], boff, zero)
    z_cat = jnp.concatenate(zs, axis=1)
    g_cat = jnp.concatenate([g[l * _G:(l + 1) * _G] for l in range(_L)], axis=1)
    return (z_cat, g_cat)


# speedup vs baseline: 1.3638x; 1.0591x over previous
import functools

import jax
import jax.numpy as jnp
import numpy as np
from jax import lax
from jax.experimental import pallas as pl
from jax.experimental.pallas import tpu as pltpu
from jax.experimental.pallas import tpu_sc as plsc

_ATOM_DIMS = [119, 5, 12, 12, 10, 6, 6, 2, 2]
_BOND_DIMS = [5, 6, 2]
_AOFF = np.concatenate([[0], np.cumsum(_ATOM_DIMS)[:-1]]).astype(np.int32)
_BOFF = np.concatenate([[0], np.cumsum(_BOND_DIMS)[:-1]]).astype(np.int32)

_N = 10000
_E = 320000
_H = 128
_L = 3
_G = 128

_NC, _NS, _NW = 2, 16, 32
_CE = 128
_NP = 10240
_CH = 80
_K0 = 144
_K1 = 106
_EP = _E
_NA = 10112
_RT = _NA // _NS

_mesh = plsc.VectorSubcoreMesh(
    core_axis_name="c", subcore_axis_name="s", num_cores=_NC, num_subcores=_NS
)


def _add_rows(dst, src, rows):
    @pl.loop(0, rows)
    def _(r):
        for j in range(_H // 16):
            sl = pl.ds(j * 16, 16)
            dst[r, sl] = dst[r, sl] + src[r, sl]


_REP = 16

@functools.partial(
    pl.kernel,
    out_type=jax.ShapeDtypeStruct((_NP, _H), jnp.float32),
    mesh=_mesh,
    scratch_types=[
        pltpu.VMEM((_CE,), jnp.int32),
        pltpu.VMEM((_CE, _H), jnp.float32),
        pltpu.VMEM((_CE, _H), jnp.float32),
    ],
)
def _encode(aidx, atab, z0, idx_v, acc_v, fld_v):
    c = lax.axis_index("c")
    s = lax.axis_index("s")
    w = c * _NS + s

    @pl.loop(0, 3)
    def _(k):
        cid = w + k * _NW

        @pl.when(cid < _NP // _CE)
        def _():
            base = cid * _CE
            pltpu.sync_copy(aidx.at[pl.ds(base, _CE)], idx_v)
            pltpu.sync_copy(atab.at[idx_v], acc_v)
            for f in range(1, 9):
                pltpu.sync_copy(aidx.at[pl.ds(f * _NP + base, _CE)], idx_v)
                pltpu.sync_copy(atab.at[idx_v], fld_v)
                _add_rows(acc_v, fld_v, _CE)
            pltpu.sync_copy(acc_v, z0.at[pl.ds(base, _CE)])


@functools.partial(
    pl.kernel,
    out_type=jax.ShapeDtypeStruct((_NC, _NA, _H), jnp.float32),
    mesh=_mesh,
    scratch_types=[
        pltpu.VMEM_SHARED((_NA, _H), jnp.float32),
        pltpu.VMEM((2, _CH), jnp.int32),
        pltpu.VMEM((2, _CH), jnp.int32),
        pltpu.VMEM((2, _CH), jnp.int32),
        pltpu.VMEM((2, _CH), jnp.int32),
        pltpu.VMEM((_CH, _H), jnp.float32),
        pltpu.VMEM((_CH, _H), jnp.float32),
        pltpu.VMEM((_CH, _H), jnp.float32),
        pltpu.VMEM((_CH, _H), jnp.float32),
        pltpu.SemaphoreType.DMA,
        pltpu.SemaphoreType.DMA,
        pltpu.SemaphoreType.DMA,
        pltpu.SemaphoreType.DMA,
        pltpu.SemaphoreType.DMA,
        pltpu.SemaphoreType.DMA,
    ],
)
def _message(z, cidx, src, dst, ctab, zero, part, acc,
             src_v, cidx_v, dst_v, dst_sc, zb0, zb1, eb0, eb1,
             si0, si1, sg0, sg1, ss0, ss1):
    c = lax.axis_index("c")
    s = lax.axis_index("s")
    zb = (zb0, zb1)
    eb = (eb0, eb1)
    si = (si0, si1)
    sg = (sg0, sg1)
    ss = (ss0, ss1)

    my_ke = jnp.where(c == 0, _K0, _K1)
    chunk0 = jnp.where(c == 0, s * _K0, _NS * _K0 + s * _K1)

    def issue_idx(k, b):
        base = (chunk0 + k) * _CH
        pltpu.async_copy(src.at[pl.ds(base, _CH)], src_v.at[b], si[b])
        pltpu.async_copy(cidx.at[pl.ds(base, _CH)], cidx_v.at[b], si[b])
        pltpu.async_copy(dst.at[pl.ds(base, _CH)], dst_v.at[b], si[b])

    def wait_idx(b):
        for ref in (src_v, cidx_v, dst_v):
            pltpu.make_async_copy(src.at[pl.ds(0, _CH)], ref.at[b], si[b]).wait()

    def issue_gathers(b):
        pltpu.async_copy(z.at[src_v.at[b]], zb[b], sg[b])
        pltpu.async_copy(ctab.at[cidx_v.at[b]], eb[b], sg[b])

    def wait_gathers(b):
        pltpu.make_async_copy(z.at[pl.ds(0, _CH)], zb[b], sg[b]).wait()
        pltpu.make_async_copy(z.at[pl.ds(0, _CH)], eb[b], sg[b]).wait()

    def issue_scatter(b):
        for j in range(_CH // 16):
            sl = pl.ds(j * 16, 16)
            dst_sc[b, sl] = dst_v[b, sl]
        pltpu.async_copy(zb[b], acc.at[dst_sc.at[b]], ss[b], add=True)

    def wait_scatter(b):
        pltpu.make_async_copy(zb[b], acc.at[pl.ds(0, _CH)], ss[b]).wait()

    def relu_add(b):
        @pl.loop(0, _CH)
        def _(r):
            for j in range(_H // 16):
                sl = pl.ds(j * 16, 16)
                zb[b][r, sl] = jnp.maximum(zb[b][r, sl] + eb[b][r, sl], 0.0)

    pltpu.sync_copy(zero, acc.at[pl.ds(s * _RT, _RT)])
    plsc.subcore_barrier()

    issue_idx(0, 0)
    wait_idx(0)
    issue_gathers(0)
    issue_idx(1, 1)

    @pl.loop(0, my_ke // 2)
    def _(t):
        k0 = t * 2
        wait_gathers(0)
        wait_idx(1)

        @pl.when(t > 0)
        def _():
            wait_scatter(1)

        issue_gathers(1)
        relu_add(0)
        issue_scatter(0)

        @pl.when(k0 + 2 < my_ke)
        def _():
            issue_idx(k0 + 2, 0)

        wait_gathers(1)
        relu_add(1)

        @pl.when(k0 + 2 < my_ke)
        def _():
            wait_idx(0)
            wait_scatter(0)
            issue_gathers(0)

        issue_scatter(1)

        @pl.when(k0 + 3 < my_ke)
        def _():
            issue_idx(k0 + 3, 1)

    wait_scatter(0)
    wait_scatter(1)
    plsc.subcore_barrier()
    pltpu.sync_copy(acc.at[pl.ds(s * _RT, _RT)], part.at[c, pl.ds(s * _RT, _RT)])


@functools.partial(
    pl.kernel,
    out_type=jax.ShapeDtypeStruct((_L * _G, _H), jnp.float32),
    mesh=_mesh,
    scratch_types=[
        pltpu.VMEM_SHARED((_L * _G, _H), jnp.float32),
        pltpu.VMEM((80,), jnp.int32),
        pltpu.VMEM((80, _H), jnp.float32),
    ],
)
def _pool(z0, z1, z2, boff, zero, g, acc, idx_v, buf):
    c = lax.axis_index("c")
    s = lax.axis_index("s")
    nrows = _L * _G // _NS

    @pl.when(c == 0)
    def _():
        pltpu.sync_copy(zero.at[pl.ds(0, nrows)], acc.at[pl.ds(s * nrows, nrows)])

    plsc.subcore_barrier()

    @pl.when(c == 0)
    def _():
        for l, zref in enumerate((z0, z1, z2)):
            @pl.loop(0, 8)
            def _(k, l=l, zref=zref):
                cid = s + k * _NS

                @pl.when(cid < _N // 80)
                def _():
                    row = cid * 80
                    pltpu.sync_copy(boff.at[pl.ds(l * _N + row, 80)], idx_v)
                    pltpu.sync_copy(zref.at[pl.ds(row, 80)], buf)
                    pltpu.sync_copy(buf, acc.at[idx_v], add=True)

    plsc.subcore_barrier()

    @pl.when(c == 0)
    def _():
        pltpu.sync_copy(acc.at[pl.ds(s * nrows, nrows)], g.at[pl.ds(s * nrows, nrows)])


def _mlp_body(final, z_ref, p_ref, w1, b1, g1, be1, w2, b2, gbn, bbn, out):
    h = z_ref[...] + p_ref[0, 0:_N, :] + p_ref[1, 0:_N, :]
    h1 = jnp.dot(h, w1[...], preferred_element_type=jnp.float32) + b1[...]
    m = jnp.mean(h1, axis=0, keepdims=True)
    d = h1 - m
    v = jnp.mean(d * d, axis=0, keepdims=True)
    h1 = d * (g1[...] * lax.rsqrt(v + 1e-5)) + be1[...]
    h1 = jnp.maximum(h1, 0.0)
    h2 = jnp.dot(h1, w2[...], preferred_element_type=jnp.float32) + b2[...]
    m2 = jnp.mean(h2, axis=0, keepdims=True)
    d2 = h2 - m2
    v2 = jnp.mean(d2 * d2, axis=0, keepdims=True)
    zn = d2 * (gbn[...] * lax.rsqrt(v2 + 1e-5)) + bbn[...]
    if not final:
        zn = jnp.maximum(zn, 0.0)
    out[...] = zn


def _mlp(final):
    return pl.pallas_call(
        functools.partial(_mlp_body, final),
        out_shape=jax.ShapeDtypeStruct((_N, _H), jnp.float32),
    )


@jax.jit
def kernel(x, edge_index, edge_attr, batch, atom_table, bond_table,
           W1, b1, g1, be1, W2, b2, gbn, bbn):
    nrep = jnp.arange(_N, dtype=jnp.int32) % _REP
    aidx = (x.astype(jnp.int32) + jnp.asarray(_AOFF)[None, :]).T
    aidx = aidx + (nrep * np.int32(sum(_ATOM_DIMS)))[None, :]
    aidx = jnp.pad(aidx, ((0, 0), (0, _NP - _N))).reshape(-1)
    atab_rep = jnp.tile(atom_table, (_REP, 1))

    bt0 = bond_table[_BOFF[0]:_BOFF[0] + _BOND_DIMS[0]]
    bt1 = bond_table[_BOFF[1]:_BOFF[1] + _BOND_DIMS[1]]
    bt2 = bond_table[_BOFF[2]:_BOFF[2] + _BOND_DIMS[2]]
    ctab = (bt0[:, None, None, :] + bt1[None, :, None, :]
            + bt2[None, None, :, :]).reshape(-1, _H)
    ctab_rep = jnp.tile(ctab, (_REP, 1))
    ea = edge_attr.astype(jnp.int32)
    ncomb = np.int32(_BOND_DIMS[1] * _BOND_DIMS[2])
    bidx = (ea[:, 0] * ncomb + ea[:, 1] * np.int32(_BOND_DIMS[2]) + ea[:, 2]
            + (jnp.arange(_E, dtype=jnp.int32) % _REP) * np.int32(60))
    bidx = jnp.pad(bidx, (0, _EP - _E))
    src = jnp.pad(edge_index[0].astype(jnp.int32), (0, _EP - _E))
    dst = jnp.pad(edge_index[1].astype(jnp.int32), (0, _EP - _E),
                  constant_values=_N)
    boff = (batch.astype(jnp.int32)[None, :]
            + (jnp.arange(_L, dtype=jnp.int32) * _G)[:, None]).reshape(-1)
    zero = jnp.zeros((_RT, _H), jnp.float32)

    z0f = _encode(aidx, atab_rep)
    z = z0f[:_N]

    zs = []
    for l in range(_L):
        part = _message(z, bidx, src, dst, ctab_rep, zero)
        z = _mlp(l == _L - 1)(
            z, part, W1[l], b1[l][None, :], g1[l][None, :], be1[l][None, :],
            W2[l], b2[l][None, :], gbn[l][None, :], bbn[l][None, :])
        zs.append(z)

    g = _pool(zs[0], zs[1], zs[2], boff, zero)
    z_cat = jnp.concatenate(zs, axis=1)
    g_cat = jnp.concatenate([g[l * _G:(l + 1) * _G] for l in range(_L)], axis=1)
    return (z_cat, g_cat)
